# baseline (speedup 1.0000x reference)
import functools

import jax
import jax.numpy as jnp
import numpy as np
from jax.experimental import pallas as pl
from jax.experimental.pallas import tpu as pltpu

EPS = 1e-9
SCORE_TH = 0.5
BIG = 100000.0


def _fused_kernel(starts, ends, n_grid_n, n_grid_a,
                  centers_t_ref, gt_ref, padm_ref, predt_ref, pred_ref,
                  scores_ref, cost_ref, w_ref, h_ref,
                  acc_cnt, acc_w, acc_h):
    n = pl.program_id(0)
    a = pl.program_id(1)
    blk = centers_t_ref.shape[1]

    cx = centers_t_ref[0:1, :]
    cy = centers_t_ref[1:2, :]
    pt = predt_ref[0]
    px1 = pt[0:1, :]
    py1 = pt[1:2, :]
    px2 = pt[2:3, :]
    py2 = pt[3:4, :]
    l_ = cx - px1
    t_ = cy - py1
    r_ = px2 - cx
    b_ = py2 - cy
    num = jnp.minimum(l_, r_) * jnp.minimum(t_, b_)
    den = jnp.maximum(l_, r_) * jnp.maximum(t_, b_)
    ratio = jnp.clip(num / jnp.maximum(den, EPS), 1e-12, 1.0)
    centerness = jnp.sqrt(ratio)

    g = gt_ref[0]
    gx1 = g[:, 0:1]
    gy1 = g[:, 1:2]
    gx2 = g[:, 2:3]
    gy2 = g[:, 3:4]
    padm = padm_ref[0]

    inside = ((cx - gx1 > EPS) & (cy - gy1 > EPS)
              & (gx2 - cx > EPS) & (gy2 - cy > EPS))
    maskf = inside.astype(jnp.float32) * padm

    ix1 = jnp.maximum(gx1, px1)
    iy1 = jnp.maximum(gy1, py1)
    ix2 = jnp.minimum(gx2, px2)
    iy2 = jnp.minimum(gy2, py2)
    inter = jnp.maximum(ix2 - ix1, 0.0) * jnp.maximum(iy2 - iy1, 0.0)
    ag = (gx2 - gx1) * (gy2 - gy1)
    ap = (px2 - px1) * (py2 - py1)
    iou = inter / (ag + ap - inter + EPS)

    inv = 1.0 - maskf
    cost_ref[0, 0] = (1.0 - centerness) * maskf + inv * BIG
    cost_ref[1, 0] = (1.0 - iou) * maskf + inv * BIG

    @pl.when((n == 0) & (a == 0))
    def _init():
        acc_cnt[...] = jnp.zeros_like(acc_cnt)
        acc_w[...] = jnp.zeros_like(acc_w)
        acc_h[...] = jnp.zeros_like(acc_h)

    s = scores_ref[0]
    flag = jnp.max(s, axis=1, keepdims=True) > SCORE_TH
    pb = pred_ref[0]
    wcol = pb[:, 2:3] - pb[:, 0:1]
    hcol = pb[:, 3:4] - pb[:, 1:2]
    gidx = a * blk + jax.lax.broadcasted_iota(jnp.int32, (blk, 1), 0)
    lev = jax.lax.broadcasted_iota(jnp.int32, (1, len(starts)), 1)
    starts_v = sum((lev == i) * s for i, s in enumerate(starts))
    ends_v = sum((lev == i) * e for i, e in enumerate(ends))
    levmask = (gidx >= starts_v) & (gidx < ends_v)
    flag4 = flag & levmask
    cntf = flag4.astype(jnp.float32)
    acc_cnt[...] += jnp.sum(cntf, axis=0, keepdims=True)
    acc_w[...] += jnp.sum(jnp.where(flag4, wcol, 0.0), axis=0, keepdims=True)
    acc_h[...] += jnp.sum(jnp.where(flag4, hcol, 0.0), axis=0, keepdims=True)

    @pl.when((n == n_grid_n - 1) & (a == n_grid_a - 1))
    def _final():
        cnt = jnp.maximum(acc_cnt[...], 1.0)
        w_ref[...] = acc_w[...] / cnt * 0.9
        h_ref[...] = acc_h[...] / cnt * 0.9


NUM_ANCHORS_LIST = (6400, 1600, 400, 100)


def _run(centers, gt_bboxes, pad_gt_mask, pred_bboxes, pred_scores):
    A = centers.shape[0]
    N, M, _ = gt_bboxes.shape
    NC = pred_scores.shape[2]
    BLK = 512
    n_a = (A + BLK - 1) // BLK

    bounds = np.concatenate([[0], np.cumsum(NUM_ANCHORS_LIST)])
    starts = tuple(int(x) for x in bounds[:-1])
    ends = tuple(int(x) for x in bounds[1:])

    centers_t = centers.T
    predt = jnp.transpose(pred_bboxes, (0, 2, 1))

    grid = (N, n_a)
    kfn = functools.partial(_fused_kernel, starts, ends, N, n_a)
    cost, w_avg, h_avg = pl.pallas_call(
        kfn,
        grid=grid,
        in_specs=[
            pl.BlockSpec((2, BLK), lambda n, a: (0, a)),
            pl.BlockSpec((1, M, 4), lambda n, a: (n, 0, 0)),
            pl.BlockSpec((1, M, 1), lambda n, a: (n, 0, 0)),
            pl.BlockSpec((1, 4, BLK), lambda n, a: (n, 0, a)),
            pl.BlockSpec((1, BLK, 4), lambda n, a: (n, a, 0)),
            pl.BlockSpec((1, BLK, NC), lambda n, a: (n, a, 0)),
        ],
        out_specs=[
            pl.BlockSpec((2, 1, M, BLK), lambda n, a: (0, n, 0, a)),
            pl.BlockSpec((1, 4), lambda n, a: (0, 0)),
            pl.BlockSpec((1, 4), lambda n, a: (0, 0)),
        ],
        out_shape=[
            jax.ShapeDtypeStruct((2, N, M, A), jnp.float32),
            jax.ShapeDtypeStruct((1, 4), jnp.float32),
            jax.ShapeDtypeStruct((1, 4), jnp.float32),
        ],
        scratch_shapes=[
            pltpu.VMEM((1, 4), jnp.float32),
            pltpu.VMEM((1, 4), jnp.float32),
            pltpu.VMEM((1, 4), jnp.float32),
        ],
        compiler_params=pltpu.CompilerParams(
            dimension_semantics=("arbitrary", "arbitrary")),
    )(centers_t, gt_bboxes, pad_gt_mask, predt, pred_bboxes, pred_scores)
    return cost, w_avg.reshape(4), h_avg.reshape(4)


def kernel(centers, num_anchors_list, gt_labels, gt_bboxes, pad_gt_mask,
           bg_index, pred_bboxes, pred_scores):
    return _run(centers, gt_bboxes, pad_gt_mask, pred_bboxes, pred_scores)


# speedup vs baseline: 1.3164x; 1.3164x over previous
import jax
import jax.numpy as jnp
import numpy as np
from jax.experimental import pallas as pl
from jax.experimental.pallas import tpu as pltpu

EPS = 1e-9
SCORE_TH = 0.5
BIG = 100000.0

NUM_ANCHORS_LIST = (6400, 1600, 400, 100)


def _fused_kernel(starts, ends, n_grid_n, n_grid_a,
                  centers_t_ref, gt_ref, padm_ref, predt_ref,
                  scores_ref, cost_ref, w_ref, h_ref,
                  acc_cnt, acc_w, acc_h):
    n = pl.program_id(0)
    a = pl.program_id(1)
    blk = centers_t_ref.shape[1]
    nlev = len(starts)

    cx = centers_t_ref[0:1, :]
    cy = centers_t_ref[1:2, :]
    pt = predt_ref[0]
    px1 = pt[0:1, :]
    py1 = pt[1:2, :]
    px2 = pt[2:3, :]
    py2 = pt[3:4, :]
    w_row = px2 - px1
    h_row = py2 - py1
    l_ = cx - px1
    t_ = cy - py1
    r_ = px2 - cx
    b_ = py2 - cy
    num = jnp.minimum(l_, r_) * jnp.minimum(t_, b_)
    den = jnp.maximum(l_, r_) * jnp.maximum(t_, b_)
    ratio = jnp.clip(num / jnp.maximum(den, EPS), 1e-12, 1.0)
    one_m_centerness = 1.0 - jnp.sqrt(ratio)

    g = gt_ref[0]
    gx1 = g[:, 0:1]
    gy1 = g[:, 1:2]
    gx2 = g[:, 2:3]
    gy2 = g[:, 3:4]
    padb = padm_ref[0] > 0.5

    d1 = cx - gx1
    d2 = cy - gy1
    d3 = gx2 - cx
    d4 = gy2 - cy
    mind = jnp.minimum(jnp.minimum(d1, d2), jnp.minimum(d3, d4))
    valid = (mind > EPS) & padb

    ix1 = jnp.maximum(gx1, px1)
    iy1 = jnp.maximum(gy1, py1)
    ix2 = jnp.minimum(gx2, px2)
    iy2 = jnp.minimum(gy2, py2)
    inter = jnp.maximum(ix2 - ix1, 0.0) * jnp.maximum(iy2 - iy1, 0.0)
    agE = (gx2 - gx1) * (gy2 - gy1) + EPS
    ap = w_row * h_row
    union = (agE + ap) - inter
    one_m_iou = 1.0 - inter / union

    cost_ref[0, 0] = jnp.where(valid, one_m_centerness, BIG)
    cost_ref[1, 0] = jnp.where(valid, one_m_iou, BIG)

    @pl.when((n == 0) & (a == 0))
    def _init():
        acc_cnt[...] = jnp.zeros_like(acc_cnt)
        acc_w[...] = jnp.zeros_like(acc_w)
        acc_h[...] = jnp.zeros_like(acc_h)

    s = scores_ref[0]
    maxv = jnp.max(s, axis=1, keepdims=True)
    flag_row = maxv.reshape(1, blk) > SCORE_TH

    gidx = a * blk + jax.lax.broadcasted_iota(jnp.int32, (1, blk), 1)
    lev = jax.lax.broadcasted_iota(jnp.int32, (nlev, 1), 0)
    starts_c = sum((lev == i) * s0 for i, s0 in enumerate(starts))
    ends_c = sum((lev == i) * e0 for i, e0 in enumerate(ends))
    levmask = (gidx >= starts_c) & (gidx < ends_c)
    flag4 = levmask & flag_row
    acc_cnt[...] += flag4.astype(jnp.float32)
    acc_w[...] += jnp.where(flag4, w_row, 0.0)
    acc_h[...] += jnp.where(flag4, h_row, 0.0)

    @pl.when((n == n_grid_n - 1) & (a == n_grid_a - 1))
    def _final():
        cnt = jnp.maximum(
            jnp.sum(acc_cnt[...], axis=1, keepdims=True), 1.0)
        w_ref[...] = jnp.sum(acc_w[...], axis=1, keepdims=True) / cnt * 0.9
        h_ref[...] = jnp.sum(acc_h[...], axis=1, keepdims=True) / cnt * 0.9


def _run(centers, gt_bboxes, pad_gt_mask, pred_bboxes, pred_scores):
    A = centers.shape[0]
    N, M, _ = gt_bboxes.shape
    NC = pred_scores.shape[2]
    BLK = 512
    n_a = (A + BLK - 1) // BLK

    bounds = np.concatenate([[0], np.cumsum(NUM_ANCHORS_LIST)])
    starts = tuple(int(x) for x in bounds[:-1])
    ends = tuple(int(x) for x in bounds[1:])

    centers_t = centers.T
    predt = jnp.transpose(pred_bboxes, (0, 2, 1))

    def kfn(*refs):
        return _fused_kernel(starts, ends, N, n_a, *refs)

    cost, w_avg, h_avg = pl.pallas_call(
        kfn,
        grid=(N, n_a),
        in_specs=[
            pl.BlockSpec((2, BLK), lambda n, a: (0, a)),
            pl.BlockSpec((1, M, 4), lambda n, a: (n, 0, 0)),
            pl.BlockSpec((1, M, 1), lambda n, a: (n, 0, 0)),
            pl.BlockSpec((1, 4, BLK), lambda n, a: (n, 0, a)),
            pl.BlockSpec((1, BLK, NC), lambda n, a: (n, a, 0)),
        ],
        out_specs=[
            pl.BlockSpec((2, 1, M, BLK), lambda n, a: (0, n, 0, a)),
            pl.BlockSpec((4, 1), lambda n, a: (0, 0)),
            pl.BlockSpec((4, 1), lambda n, a: (0, 0)),
        ],
        out_shape=[
            jax.ShapeDtypeStruct((2, N, M, A), jnp.float32),
            jax.ShapeDtypeStruct((4, 1), jnp.float32),
            jax.ShapeDtypeStruct((4, 1), jnp.float32),
        ],
        scratch_shapes=[
            pltpu.VMEM((4, BLK), jnp.float32),
            pltpu.VMEM((4, BLK), jnp.float32),
            pltpu.VMEM((4, BLK), jnp.float32),
        ],
        compiler_params=pltpu.CompilerParams(
            dimension_semantics=("arbitrary", "arbitrary")),
    )(centers_t, gt_bboxes, pad_gt_mask, predt, pred_scores)
    return cost, w_avg.reshape(4), h_avg.reshape(4)


def kernel(centers, num_anchors_list, gt_labels, gt_bboxes, pad_gt_mask,
           bg_index, pred_bboxes, pred_scores):
    return _run(centers, gt_bboxes, pad_gt_mask, pred_bboxes, pred_scores)


# speedup vs baseline: 1.6466x; 1.2509x over previous
import jax
import jax.numpy as jnp
import numpy as np
from jax.experimental import pallas as pl
from jax.experimental.pallas import tpu as pltpu

EPS = 1e-9
SCORE_TH = 0.5
BIG = 100000.0

NUM_ANCHORS_LIST = (6400, 1600, 400, 100)


def _fused_kernel(starts, ends, n_grid_n, n_grid_a,
                  centers_t_ref, gt_ref, padm_ref, predt_ref,
                  scores_ref, cost_ref, w_ref, h_ref,
                  acc_cnt, acc_w, acc_h):
    n = pl.program_id(0)
    a = pl.program_id(1)
    blk = centers_t_ref.shape[1]
    nlev = len(starts)

    cx = centers_t_ref[0:1, :]
    cy = centers_t_ref[1:2, :]
    pt = predt_ref[0]
    px1 = pt[0:1, :]
    py1 = pt[1:2, :]
    px2 = pt[2:3, :]
    py2 = pt[3:4, :]
    w_row = px2 - px1
    h_row = py2 - py1
    l_ = cx - px1
    t_ = cy - py1
    r_ = px2 - cx
    b_ = py2 - cy
    num = jnp.minimum(l_, r_) * jnp.minimum(t_, b_)
    den = jnp.maximum(l_, r_) * jnp.maximum(t_, b_)
    ratio = jnp.clip(num / jnp.maximum(den, EPS), 1e-12, 1.0)
    one_m_centerness = 1.0 - jnp.sqrt(ratio)

    g = gt_ref[0]
    gx1 = g[:, 0:1]
    gy1 = g[:, 1:2]
    gx2 = g[:, 2:3]
    gy2 = g[:, 3:4]
    padb = padm_ref[0] > 0.5

    d1 = cx - gx1
    d2 = cy - gy1
    d3 = gx2 - cx
    d4 = gy2 - cy
    mind = jnp.minimum(jnp.minimum(d1, d2), jnp.minimum(d3, d4))
    valid = (mind > EPS) & padb

    ix1 = jnp.maximum(gx1, px1)
    iy1 = jnp.maximum(gy1, py1)
    ix2 = jnp.minimum(gx2, px2)
    iy2 = jnp.minimum(gy2, py2)
    inter = jnp.maximum(ix2 - ix1, 0.0) * jnp.maximum(iy2 - iy1, 0.0)
    agE = (gx2 - gx1) * (gy2 - gy1) + EPS
    ap = w_row * h_row
    union = (agE + ap) - inter
    one_m_iou = 1.0 - inter / union

    cost_ref[0, 0] = jnp.where(valid, one_m_centerness, BIG)
    cost_ref[1, 0] = jnp.where(valid, one_m_iou, BIG)

    @pl.when((n == 0) & (a == 0))
    def _init():
        acc_cnt[...] = jnp.zeros_like(acc_cnt)
        acc_w[...] = jnp.zeros_like(acc_w)
        acc_h[...] = jnp.zeros_like(acc_h)

    s = scores_ref[0]
    maxv = jnp.max(s, axis=1, keepdims=True)
    flag_row = maxv.reshape(1, blk) > SCORE_TH

    gidx = a * blk + jax.lax.broadcasted_iota(jnp.int32, (1, blk), 1)
    lev = jax.lax.broadcasted_iota(jnp.int32, (nlev, 1), 0)
    starts_c = sum((lev == i) * s0 for i, s0 in enumerate(starts))
    ends_c = sum((lev == i) * e0 for i, e0 in enumerate(ends))
    levmask = (gidx >= starts_c) & (gidx < ends_c)
    flag4 = levmask & flag_row
    acc_cnt[...] += flag4.astype(jnp.float32)
    acc_w[...] += jnp.where(flag4, w_row, 0.0)
    acc_h[...] += jnp.where(flag4, h_row, 0.0)

    @pl.when((n == n_grid_n - 1) & (a == n_grid_a - 1))
    def _final():
        cnt = jnp.maximum(
            jnp.sum(acc_cnt[...], axis=1, keepdims=True), 1.0)
        w_ref[...] = jnp.sum(acc_w[...], axis=1, keepdims=True) / cnt * 0.9
        h_ref[...] = jnp.sum(acc_h[...], axis=1, keepdims=True) / cnt * 0.9


def _run(centers, gt_bboxes, pad_gt_mask, pred_bboxes, pred_scores):
    A = centers.shape[0]
    N, M, _ = gt_bboxes.shape
    NC = pred_scores.shape[2]
    BLK = 1024
    n_a = (A + BLK - 1) // BLK

    bounds = np.concatenate([[0], np.cumsum(NUM_ANCHORS_LIST)])
    starts = tuple(int(x) for x in bounds[:-1])
    ends = tuple(int(x) for x in bounds[1:])

    centers_t = centers.T
    predt = jnp.transpose(pred_bboxes, (0, 2, 1))

    def kfn(*refs):
        return _fused_kernel(starts, ends, N, n_a, *refs)

    cost, w_avg, h_avg = pl.pallas_call(
        kfn,
        grid=(N, n_a),
        in_specs=[
            pl.BlockSpec((2, BLK), lambda n, a: (0, a)),
            pl.BlockSpec((1, M, 4), lambda n, a: (n, 0, 0)),
            pl.BlockSpec((1, M, 1), lambda n, a: (n, 0, 0)),
            pl.BlockSpec((1, 4, BLK), lambda n, a: (n, 0, a)),
            pl.BlockSpec((1, BLK, NC), lambda n, a: (n, a, 0)),
        ],
        out_specs=[
            pl.BlockSpec((2, 1, M, BLK), lambda n, a: (0, n, 0, a)),
            pl.BlockSpec((4, 1), lambda n, a: (0, 0)),
            pl.BlockSpec((4, 1), lambda n, a: (0, 0)),
        ],
        out_shape=[
            jax.ShapeDtypeStruct((2, N, M, A), jnp.float32),
            jax.ShapeDtypeStruct((4, 1), jnp.float32),
            jax.ShapeDtypeStruct((4, 1), jnp.float32),
        ],
        scratch_shapes=[
            pltpu.VMEM((4, BLK), jnp.float32),
            pltpu.VMEM((4, BLK), jnp.float32),
            pltpu.VMEM((4, BLK), jnp.float32),
        ],
        compiler_params=pltpu.CompilerParams(
            dimension_semantics=("arbitrary", "arbitrary")),
    )(centers_t, gt_bboxes, pad_gt_mask, predt, pred_scores)
    return cost, w_avg.reshape(4), h_avg.reshape(4)


def kernel(centers, num_anchors_list, gt_labels, gt_bboxes, pad_gt_mask,
           bg_index, pred_bboxes, pred_scores):
    return _run(centers, gt_bboxes, pad_gt_mask, pred_bboxes, pred_scores)


# speedup vs baseline: 1.9489x; 1.1836x over previous
import jax
import jax.numpy as jnp
import numpy as np
from jax.experimental import pallas as pl
from jax.experimental.pallas import tpu as pltpu

EPS = 1e-9
SCORE_TH = 0.5
BIG = 100000.0

NUM_ANCHORS_LIST = (6400, 1600, 400, 100)


def _fused_kernel(starts, ends, n_grid_n, n_grid_a,
                  centers_t_ref, gt_ref, padm_ref, predt_ref,
                  scores_ref, cost_ref, w_ref, h_ref,
                  acc_cnt, acc_w, acc_h):
    n = pl.program_id(0)
    a = pl.program_id(1)
    blk = centers_t_ref.shape[1]
    nlev = len(starts)

    cx = centers_t_ref[0:1, :]
    cy = centers_t_ref[1:2, :]
    pt = predt_ref[0]
    px1 = pt[0:1, :]
    py1 = pt[1:2, :]
    px2 = pt[2:3, :]
    py2 = pt[3:4, :]
    w_row = px2 - px1
    h_row = py2 - py1
    l_ = cx - px1
    t_ = cy - py1
    r_ = px2 - cx
    b_ = py2 - cy
    num = jnp.minimum(l_, r_) * jnp.minimum(t_, b_)
    den = jnp.maximum(l_, r_) * jnp.maximum(t_, b_)
    ratio = jnp.clip(num / jnp.maximum(den, EPS), 1e-12, 1.0)
    one_m_centerness = 1.0 - jnp.sqrt(ratio)

    g = gt_ref[0]
    gx1 = g[:, 0:1]
    gy1 = g[:, 1:2]
    gx2 = g[:, 2:3]
    gy2 = g[:, 3:4]
    padb = padm_ref[0] > 0.5

    d1 = cx - gx1
    d2 = cy - gy1
    d3 = gx2 - cx
    d4 = gy2 - cy
    mind = jnp.minimum(jnp.minimum(d1, d2), jnp.minimum(d3, d4))
    valid = (mind > EPS) & padb

    ix1 = jnp.maximum(gx1, px1)
    iy1 = jnp.maximum(gy1, py1)
    ix2 = jnp.minimum(gx2, px2)
    iy2 = jnp.minimum(gy2, py2)
    inter = jnp.maximum(ix2 - ix1, 0.0) * jnp.maximum(iy2 - iy1, 0.0)
    agE = (gx2 - gx1) * (gy2 - gy1) + EPS
    ap = w_row * h_row
    union = (agE + ap) - inter
    one_m_iou = 1.0 - inter / union

    cost_ref[0, 0] = jnp.where(valid, one_m_centerness, BIG)
    cost_ref[1, 0] = jnp.where(valid, one_m_iou, BIG)

    @pl.when((n == 0) & (a == 0))
    def _init():
        acc_cnt[...] = jnp.zeros_like(acc_cnt)
        acc_w[...] = jnp.zeros_like(acc_w)
        acc_h[...] = jnp.zeros_like(acc_h)

    s = scores_ref[0]
    maxv = jnp.max(s, axis=1, keepdims=True)
    flag_row = maxv.reshape(1, blk) > SCORE_TH

    gidx = a * blk + jax.lax.broadcasted_iota(jnp.int32, (1, blk), 1)
    lev = jax.lax.broadcasted_iota(jnp.int32, (nlev, 1), 0)
    starts_c = sum((lev == i) * s0 for i, s0 in enumerate(starts))
    ends_c = sum((lev == i) * e0 for i, e0 in enumerate(ends))
    levmask = (gidx >= starts_c) & (gidx < ends_c)
    flag4 = levmask & flag_row
    acc_cnt[...] += flag4.astype(jnp.float32)
    acc_w[...] += jnp.where(flag4, w_row, 0.0)
    acc_h[...] += jnp.where(flag4, h_row, 0.0)

    @pl.when((n == n_grid_n - 1) & (a == n_grid_a - 1))
    def _final():
        cnt = jnp.maximum(
            jnp.sum(acc_cnt[...], axis=1, keepdims=True), 1.0)
        w_ref[...] = jnp.sum(acc_w[...], axis=1, keepdims=True) / cnt * 0.9
        h_ref[...] = jnp.sum(acc_h[...], axis=1, keepdims=True) / cnt * 0.9


def _run(centers, gt_bboxes, pad_gt_mask, pred_bboxes, pred_scores):
    A = centers.shape[0]
    N, M, _ = gt_bboxes.shape
    NC = pred_scores.shape[2]
    BLK = 2176
    n_a = (A + BLK - 1) // BLK

    bounds = np.concatenate([[0], np.cumsum(NUM_ANCHORS_LIST)])
    starts = tuple(int(x) for x in bounds[:-1])
    ends = tuple(int(x) for x in bounds[1:])

    centers_t = centers.T
    predt = jnp.transpose(pred_bboxes, (0, 2, 1))

    def kfn(*refs):
        return _fused_kernel(starts, ends, N, n_a, *refs)

    cost, w_avg, h_avg = pl.pallas_call(
        kfn,
        grid=(N, n_a),
        in_specs=[
            pl.BlockSpec((2, BLK), lambda n, a: (0, a)),
            pl.BlockSpec((1, M, 4), lambda n, a: (n, 0, 0)),
            pl.BlockSpec((1, M, 1), lambda n, a: (n, 0, 0)),
            pl.BlockSpec((1, 4, BLK), lambda n, a: (n, 0, a)),
            pl.BlockSpec((1, BLK, NC), lambda n, a: (n, a, 0)),
        ],
        out_specs=[
            pl.BlockSpec((2, 1, M, BLK), lambda n, a: (0, n, 0, a)),
            pl.BlockSpec((4, 1), lambda n, a: (0, 0)),
            pl.BlockSpec((4, 1), lambda n, a: (0, 0)),
        ],
        out_shape=[
            jax.ShapeDtypeStruct((2, N, M, A), jnp.float32),
            jax.ShapeDtypeStruct((4, 1), jnp.float32),
            jax.ShapeDtypeStruct((4, 1), jnp.float32),
        ],
        scratch_shapes=[
            pltpu.VMEM((4, BLK), jnp.float32),
            pltpu.VMEM((4, BLK), jnp.float32),
            pltpu.VMEM((4, BLK), jnp.float32),
        ],
        compiler_params=pltpu.CompilerParams(
            dimension_semantics=("arbitrary", "arbitrary")),
    )(centers_t, gt_bboxes, pad_gt_mask, predt, pred_scores)
    return cost, w_avg.reshape(4), h_avg.reshape(4)


def kernel(centers, num_anchors_list, gt_labels, gt_bboxes, pad_gt_mask,
           bg_index, pred_bboxes, pred_scores):
    return _run(centers, gt_bboxes, pad_gt_mask, pred_bboxes, pred_scores)
